# baseline (speedup 1.0000x reference)
import functools

import jax
import jax.numpy as jnp
from jax import lax
from jax.experimental import pallas as pl
from jax.experimental.pallas import tpu as pltpu
from jax.experimental.pallas import tpu_sc as plsc

B = 4
N = 4096
T = 1024
NH = 16
L = 16
NC = 2
NS = 16
NW = NC * NS
QPW = (2 * T) // NW
BLK = 16
NB = N // BLK
NBV = NB // L
BT = 128
SCAP = N + L
INF = float("inf")


def _d2t_body(coords_ref, d2t_ref, bm_ref):
    c0 = coords_ref[0, 0]
    c1 = coords_ref[1, 0]
    d2 = c0 * c0 + c1 * c1
    bm = jnp.min(d2.reshape(NB, BLK, BT), axis=1)
    d2t_ref[0] = d2.T.reshape(BT, N // 128, 128)
    bm_ref[0] = bm.T.reshape(BT, NB // 128, 128)


def _tc_d2t(coords, b0):
    return pl.pallas_call(
        _d2t_body,
        grid=(2, T // BT),
        in_specs=[pl.BlockSpec((2, 1, N, BT),
                               lambda b, t: (0, b0 + b, 0, t))],
        out_specs=[
            pl.BlockSpec((1, BT, N // 128, 128), lambda b, t: (b, t, 0, 0)),
            pl.BlockSpec((1, BT, NB // 128, 128), lambda b, t: (b, t, 0, 0)),
        ],
        out_shape=[
            jax.ShapeDtypeStruct((2, T, N // 128, 128), jnp.float32),
            jax.ShapeDtypeStruct((2, T, NB // 128, 128), jnp.float32),
        ],
    )(coords)


def _sc_body(b0, x_hbm, d2t_hbm, bm_hbm, q_hbm, sp_hbm,
             xb, rowring, bmall, blkb, bufd, bufn, qbuf, sbuf, sem0, sem1):
    wid = lax.axis_index("s") * NC + lax.axis_index("c")
    b = wid // 16
    q0 = (wid % 16) * QPW

    pltpu.sync_copy(x_hbm.at[pl.ds((b0 + b) * N, N)], xb)
    pltpu.sync_copy(bm_hbm.at[b, pl.ds(q0, QPW)], bmall)

    sems = (sem0, sem1)
    lanes = lax.iota(jnp.int32, L)
    inf_vec = jnp.full((L,), INF, dtype=jnp.float32)
    zero_i = jnp.zeros((L,), jnp.int32)

    for j in range((NB + L) // L):
        blkb[pl.ds(j * L, L)] = zero_i

    def start_row(t, par):
        return pltpu.async_copy(
            d2t_hbm.at[b, q0 + t], rowring.at[par], sems[par])

    def wait_row(par):
        pltpu.make_async_copy(
            d2t_hbm.at[b, 0], rowring.at[par], sems[par]).wait()

    start_row(0, 0)
    start_row(1, 1)

    def per_target(t, par, sacc):
        mk, _ = plsc.sort_key_val(bmall[t, 0, pl.ds(0, L)], lanes)

        def thr_merge(j, mk):
            ck, _ = plsc.sort_key_val(
                bmall[t, j // 8, pl.ds((j % 8) * L, L)], lanes)
            nk = jnp.minimum(mk, lax.rev(ck, (0,)))
            sk, _ = plsc.sort_key_val(nk, lanes)
            return sk

        mk = lax.fori_loop(1, NBV, thr_merge, mk)
        thr = jnp.max(mk)

        na = jnp.int32(0)
        for k in range(NBV):
            bmv = bmall[t, k // 8, pl.ds((k % 8) * L, L)]
            msk = bmv <= thr
            ones = jnp.where(msk, 1, 0).astype(jnp.int32)
            cs = plsc.cumsum(ones)
            pos = jnp.maximum(na + cs - 1, 0)
            plsc.store_scatter(blkb, [pos], k * L + lanes, mask=msk)
            na = na + jnp.sum(ones)

        ngrp = (na + L - 1) // L

        def grp_body(gi, cnt):
            blks = blkb[pl.ds(gi * L, L)]
            valid = (gi * L + lanes) < na
            base = blks * BLK
            row = blks // 8
            colb = (blks % 8) * BLK
            for k in range(BLK):
                d2v = plsc.load_gather(rowring.at[par], [row, colb + k])
                smsk = (d2v <= thr) & valid
                ones = jnp.where(smsk, 1, 0).astype(jnp.int32)
                cs = plsc.cumsum(ones)
                pos = jnp.maximum(cnt + cs - 1, 0)
                plsc.store_scatter(bufd, [pos], d2v, mask=smsk)
                plsc.store_scatter(bufn, [pos], base + k, mask=smsk)
                cnt = cnt + jnp.sum(ones)
            return cnt

        cnt = lax.fori_loop(0, ngrp, grp_body, jnp.int32(0))

        plsc.store_scatter(bufd, [cnt + lanes], inf_vec)
        plsc.store_scatter(bufn, [cnt + lanes], zero_i)

        bk, bv = plsc.sort_key_val(bufd[pl.ds(0, L)], bufn[pl.ds(0, L)])
        nch = (cnt + L - 1) // L

        def merge_body(j, carry):
            mk2, mv2 = carry
            ck, cv = plsc.sort_key_val(bufd[pl.ds(j * L, L)],
                                       bufn[pl.ds(j * L, L)])
            rk = lax.rev(ck, (0,))
            rv = lax.rev(cv, (0,))
            keep = mk2 <= rk
            nk = jnp.where(keep, mk2, rk)
            nv = jnp.where(keep, mv2, rv)
            sk, sv = plsc.sort_key_val(nk, nv)
            return (sk, sv)

        bk, bv = lax.fori_loop(1, nch, merge_body, (bk, bv))

        xg = plsc.load_gather(xb, [bv])
        w = jnp.float32(1.0) / jnp.maximum(bk, jnp.float32(1e-30))
        qbuf[pl.ds(t * NH, NH)] = xg * w
        return sacc + w

    def pair_body(tp, sacc):
        t0 = tp * 2
        wait_row(0)
        sacc = per_target(t0, 0, sacc)

        @pl.when(t0 + 2 < QPW)
        def _():
            start_row(t0 + 2, 0)

        wait_row(1)
        sacc = per_target(t0 + 1, 1, sacc)

        @pl.when(t0 + 3 < QPW)
        def _():
            start_row(t0 + 3, 1)
        return sacc

    sacc = lax.fori_loop(0, QPW // 2, pair_body,
                         jnp.zeros((L,), jnp.float32))

    sbuf[...] = sacc
    pltpu.sync_copy(qbuf, q_hbm.at[pl.ds((b * T + q0) * NH, QPW * NH)])
    pltpu.sync_copy(sbuf, sp_hbm.at[pl.ds(wid * NH, NH)])


def _sc_topk(xflat, d2t, bm, b0):
    mesh = plsc.VectorSubcoreMesh(core_axis_name="c", subcore_axis_name="s")
    fn = functools.partial(
        pl.kernel, mesh=mesh,
        compiler_params=pltpu.CompilerParams(
            needs_layout_passes=False,
            use_tc_tiling_on_sc=False,
        ),
        out_type=(
            jax.ShapeDtypeStruct((2 * T * NH,), jnp.float32),
            jax.ShapeDtypeStruct((NW * NH,), jnp.float32),
        ),
        scratch_types=[
            pltpu.VMEM((N,), jnp.float32),
            pltpu.VMEM((2, N // 128, 128), jnp.float32),
            pltpu.VMEM((QPW, NB // 128, 128), jnp.float32),
            pltpu.VMEM((NB + L,), jnp.int32),
            pltpu.VMEM((SCAP,), jnp.float32),
            pltpu.VMEM((SCAP,), jnp.int32),
            pltpu.VMEM((QPW * NH,), jnp.float32),
            pltpu.VMEM((NH,), jnp.float32),
            pltpu.SemaphoreType.DMA,
            pltpu.SemaphoreType.DMA,
        ],
    )(functools.partial(_sc_body, b0))
    return fn(xflat, d2t, bm)


def _combine_body(q_ref, sp_ref, out_ref):
    sp = sp_ref[...]
    rows = [jnp.sum(sp[16 * bb:16 * bb + 16, :], axis=0, keepdims=True)
            for bb in range(B)]
    s = jnp.concatenate(rows, axis=0)
    arec = jnp.float32(1.0) / s
    r4 = lax.broadcasted_iota(jnp.int32, (T, B), 0)
    c4 = lax.broadcasted_iota(jnp.int32, (T, B), 1)
    i4 = ((r4 // 16) % 4 == c4).astype(jnp.float32)
    p1 = jnp.dot(i4, arec, precision=jax.lax.Precision.HIGHEST)
    rt = lax.broadcasted_iota(jnp.int32, (T, NH), 0)
    ck = lax.broadcasted_iota(jnp.int32, (T, NH), 1)
    k16 = (rt % NH == ck).astype(jnp.float32)
    wcol = jnp.sum(k16 * p1, axis=1, keepdims=True)
    ru = lax.broadcasted_iota(jnp.int32, (64, T), 0)
    ct = lax.broadcasted_iota(jnp.int32, (64, T), 1)
    e = (ct // NH == ru).astype(jnp.float32)
    for bb in range(B):
        z = q_ref[bb] * wcol
        out_ref[bb] = jnp.dot(e, z, precision=jax.lax.Precision.HIGHEST)


def _combine(q, sp):
    return pl.pallas_call(
        _combine_body,
        out_shape=jax.ShapeDtypeStruct((B, 64, NH), jnp.float32),
    )(q, sp)


def kernel(x, coords_rel):
    xflat = x.reshape(B * N)
    d2t01, bm01 = _tc_d2t(coords_rel, 0)
    d2t23, bm23 = _tc_d2t(coords_rel, 2)
    q01, sp01 = _sc_topk(xflat, d2t01, bm01, 0)
    q23, sp23 = _sc_topk(xflat, d2t23, bm23, 2)
    q = jnp.concatenate([q01.reshape(2, T, NH), q23.reshape(2, T, NH)])
    sp = jnp.concatenate([sp01.reshape(NW, NH), sp23.reshape(NW, NH)])
    r = _combine(q, sp)
    return r.transpose(0, 2, 1).reshape(B, T, 1)
